# Initial kernel scaffold; baseline (speedup 1.0000x reference)
import jax
import jax.numpy as jnp
from jax.experimental import pallas as pl


def kernel(z, emb_weight):
    raise NotImplementedError("write your pallas kernel here")


# speedup vs baseline: 1.1240x; 1.1240x over previous
import jax
import jax.numpy as jnp
from jax.experimental import pallas as pl
from jax.experimental.pallas import tpu as pltpu
from jax.experimental.pallas import tpu_sc as plsc

N = 8192
K = 8192
D = 256
TOK_BLOCK = 256
COMMIT = 0.25


def _dist_argmin_kernel(z_ref, et_ref, zsq_ref, esq_ref, idx_ref, msum_ref):
    c = jax.lax.dot_general(
        z_ref[...], et_ref[...], (((1,), (0,)), ((), ())))
    d = (zsq_ref[...] + esq_ref[...]) - 2.0 * c
    m = jnp.min(d, axis=1, keepdims=True)
    iota = jax.lax.broadcasted_iota(jnp.int32, d.shape, 1)
    idx = jnp.min(jnp.where(d == m, iota, K), axis=1)
    idx_ref[...] = idx
    part = jnp.sum(m, axis=(0, 1), keepdims=True)

    @pl.when(pl.program_id(0) == 0)
    def _():
        msum_ref[...] = part

    @pl.when(pl.program_id(0) != 0)
    def _():
        msum_ref[...] += part


def _dist_argmin(z, et, zsq, esq):
    B = TOK_BLOCK
    return pl.pallas_call(
        _dist_argmin_kernel,
        grid=(N // B,),
        in_specs=[
            pl.BlockSpec((B, D), lambda i: (i, 0)),
            pl.BlockSpec((D, K), lambda i: (0, 0)),
            pl.BlockSpec((B, 1), lambda i: (i, 0)),
            pl.BlockSpec((K,), lambda i: (0,)),
        ],
        out_specs=[
            pl.BlockSpec((B,), lambda i: (i,)),
            pl.BlockSpec((1, 1), lambda i: (0, 0)),
        ],
        out_shape=[
            jax.ShapeDtypeStruct((N,), jnp.int32),
            jax.ShapeDtypeStruct((1, 1), jnp.float32),
        ],
    )(z, et, zsq, esq)


def _sc_gather(emb_weight, idx):
    mesh = plsc.VectorSubcoreMesh(
        core_axis_name="core", subcore_axis_name="subcore")
    idx2 = idx.reshape(1, N)
    W = 128

    @pl.kernel(out_type=jax.ShapeDtypeStruct((N, D), emb_weight.dtype),
               mesh=mesh)
    def gather_kernel(emb_hbm, i_hbm, o_hbm):
        def body(i_vmem, o_vmem):
            pltpu.sync_copy(emb_hbm.at[i_vmem.at[0]], o_vmem)

        pltpu.emit_pipeline(
            body,
            grid=(N // W,),
            in_specs=[pl.BlockSpec((1, W), index_map=lambda i: (0, i))],
            out_specs=[pl.BlockSpec((W, D), index_map=lambda i: (i, 0))],
            core_axis_name=("core", "subcore"),
            dimension_semantics=(pltpu.PARALLEL,),
        )(i_hbm, o_hbm)

    return gather_kernel(emb_weight, idx2)


def kernel(z, emb_weight):
    zsq = jnp.sum(z * z, axis=1, keepdims=True)
    esq = jnp.sum(emb_weight * emb_weight, axis=1)
    et = emb_weight.T
    indices, msum = _dist_argmin(z, et, zsq, esq)
    quantized = _sc_gather(emb_weight, indices)
    quantized_st = z + jax.lax.stop_gradient(quantized - z)
    loss = COMMIT * (msum[0, 0] / float(N * D))
    return (quantized_st, indices, loss)
